# Initial kernel scaffold; baseline (speedup 1.0000x reference)
import jax
import jax.numpy as jnp
from jax.experimental import pallas as pl


def kernel(xyz_sampled, t, aabb, alpha_volume):
    raise NotImplementedError("write your pallas kernel here")


# speedup vs baseline: 4.4013x; 4.4013x over previous
import functools

import jax
import jax.numpy as jnp
from jax import lax
from jax.experimental import pallas as pl
from jax.experimental.pallas import tpu as pltpu
from jax.experimental.pallas import tpu_sc as plsc

GRIDN = 128
TSZ = 16
NPTS = 1048576

NC = 2
NS = 16
NW = NC * NS
L = 16

PPW = NPTS // NW
C = 2048
NCHUNK = PPW // C
ROWS = C // 128


def _corner_offsets(i0, i1, stride):
    return i0 * stride, i1 * stride


def _axis_index_weight(vals, a0, iv):
    c = (vals - a0) * iv - jnp.float32(1.0)
    ii = (c + jnp.float32(1.0)) * jnp.float32(0.5) * jnp.float32(GRIDN - 1)
    i0i = ii.astype(jnp.int32)
    w = ii - i0i.astype(jnp.float32)
    i0 = jnp.minimum(jnp.maximum(i0i, 0), GRIDN - 1)
    i1 = jnp.minimum(i0i + 1, GRIDN - 1)
    i1 = jnp.maximum(i1, 0)
    return i0, i1, w


def _t_channel(tvals):
    v = (tvals + jnp.float32(1.0)) * jnp.float32(0.5) * jnp.float32(TSZ - 1)
    f = v.astype(jnp.int32)
    ff = f.astype(jnp.float32)
    d = v - ff
    half = jnp.float32(0.5)
    bump = jnp.where(d > half, 1, jnp.where(d == half, f & 1, 0))
    r = f + bump
    return jnp.minimum(jnp.maximum(r, 0), TSZ - 1)


def _body(vol_h, x_h, y_h, z_h, t_h, consts_h, out_h,
          consts_v, xv, yv, zv, tv, wxv, wyv, wzv,
          i000, i001, i010, i011, i100, i101, i110, i111,
          g000, g001, g010, g011, g100, g101, g110, g111,
          outv, gsem):
    cid = lax.axis_index("c")
    sid = lax.axis_index("s")
    wid = sid * NC + cid

    pltpu.sync_copy(consts_h, consts_v)
    a0x = consts_v[0]
    a0y = consts_v[1]
    a0z = consts_v[2]
    ivx = consts_v[3]
    ivy = consts_v[4]
    ivz = consts_v[5]

    idx_refs = (i000, i001, i010, i011, i100, i101, i110, i111)
    val_refs = (g000, g001, g010, g011, g100, g101, g110, g111)

    def chunk_body(ci, carry):
        base = wid * PPW + ci * C
        pltpu.sync_copy(x_h.at[pl.ds(base, C)], xv)
        pltpu.sync_copy(y_h.at[pl.ds(base, C)], yv)
        pltpu.sync_copy(z_h.at[pl.ds(base, C)], zv)
        pltpu.sync_copy(t_h.at[pl.ds(base, C)], tv)

        def pass1(r, c1):
            for k in range(8):
                off = r * 128 + k * L
                sl = pl.ds(off, L)
                xg = xv[sl]
                yg = yv[sl]
                zg = zv[sl]
                tg = tv[sl]
                ix0, ix1, wx = _axis_index_weight(xg, a0x, ivx)
                iy0, iy1, wy = _axis_index_weight(yg, a0y, ivy)
                iz0, iz1, wz = _axis_index_weight(zg, a0z, ivz)
                tt = _t_channel(tg)
                wxv[sl] = wx
                wyv[sl] = wy
                wzv[sl] = wz
                sx0, sx1 = _corner_offsets(ix0, ix1, TSZ)
                sy0, sy1 = _corner_offsets(iy0, iy1, GRIDN * TSZ)
                sz0, sz1 = _corner_offsets(iz0, iz1, GRIDN * GRIDN * TSZ)
                i000[sl] = sz0 + sy0 + sx0 + tt
                i001[sl] = sz0 + sy0 + sx1 + tt
                i010[sl] = sz0 + sy1 + sx0 + tt
                i011[sl] = sz0 + sy1 + sx1 + tt
                i100[sl] = sz1 + sy0 + sx0 + tt
                i101[sl] = sz1 + sy0 + sx1 + tt
                i110[sl] = sz1 + sy1 + sx0 + tt
                i111[sl] = sz1 + sy1 + sx1 + tt
            return c1

        lax.fori_loop(0, ROWS, pass1, 0)

        handles = []
        for iref, gref in zip(idx_refs, val_refs):
            handles.append(pltpu.async_copy(vol_h.at[iref], gref, gsem))
        for h in handles:
            h.wait()

        def pass2(r, c2):
            for k in range(8):
                off = r * 128 + k * L
                sl = pl.ds(off, L)
                wx = wxv[sl]
                wy = wyv[sl]
                wz = wzv[sl]
                v000 = g000[sl]
                v001 = g001[sl]
                v010 = g010[sl]
                v011 = g011[sl]
                v100 = g100[sl]
                v101 = g101[sl]
                v110 = g110[sl]
                v111 = g111[sl]
                c00 = v000 + wx * (v001 - v000)
                c01 = v010 + wx * (v011 - v010)
                c10 = v100 + wx * (v101 - v100)
                c11 = v110 + wx * (v111 - v110)
                c0 = c00 + wy * (c01 - c00)
                c1 = c10 + wy * (c11 - c10)
                outv[sl] = c0 + wz * (c1 - c0)
            return c2

        lax.fori_loop(0, ROWS, pass2, 0)
        pltpu.sync_copy(outv, out_h.at[pl.ds(base, C)])
        return carry

    lax.fori_loop(0, NCHUNK, chunk_body, 0)


@jax.jit
def _run(vol_flat, x, y, z, t, consts):
    mesh = plsc.VectorSubcoreMesh(
        core_axis_name="c", subcore_axis_name="s", num_cores=NC, num_subcores=NS
    )
    f = pl.kernel(
        _body,
        out_type=jax.ShapeDtypeStruct((NPTS,), jnp.float32),
        mesh=mesh,
        scratch_types=[
            pltpu.VMEM((8, L), jnp.float32),
            pltpu.VMEM((C,), jnp.float32),
            pltpu.VMEM((C,), jnp.float32),
            pltpu.VMEM((C,), jnp.float32),
            pltpu.VMEM((C,), jnp.float32),
            pltpu.VMEM((C,), jnp.float32),
            pltpu.VMEM((C,), jnp.float32),
            pltpu.VMEM((C,), jnp.float32),
        ]
        + [pltpu.VMEM((C,), jnp.int32) for _ in range(8)]
        + [pltpu.VMEM((C,), jnp.float32) for _ in range(8)]
        + [
            pltpu.VMEM((C,), jnp.float32),
            pltpu.SemaphoreType.DMA,
        ],
    )
    return f(vol_flat, x, y, z, t, consts)


def kernel(xyz_sampled, t, aabb, alpha_volume):
    a0 = aabb[0]
    iv = jnp.float32(1.0) / (aabb[1] - aabb[0]) * jnp.float32(2.0)
    consts = jnp.broadcast_to(
        jnp.concatenate([a0, iv, jnp.zeros((2,), jnp.float32)])[:, None], (8, L)
    )
    vol_flat = alpha_volume.reshape(-1)
    xt = xyz_sampled.T
    return _run(vol_flat, xt[0], xt[1], xt[2], t, consts)


# speedup vs baseline: 11.8257x; 2.6869x over previous
import functools

import jax
import jax.numpy as jnp
from jax import lax
from jax.experimental import pallas as pl
from jax.experimental.pallas import tpu as pltpu
from jax.experimental.pallas import tpu_sc as plsc

GRIDN = 128
TSZ = 16
NPTS = 1048576

NC = 2
NS = 16
NW = NC * NS
L = 16

PPW = NPTS // NW
C = 2048
NCHUNK = PPW // C
ROWS = C // 128


def _corner_offsets(i0, i1, stride):
    return i0 * stride, i1 * stride


def _axis_index_weight(vals, a0, iv):
    c = (vals - a0) * iv - jnp.float32(1.0)
    ii = (c + jnp.float32(1.0)) * jnp.float32(0.5) * jnp.float32(GRIDN - 1)
    i0i = ii.astype(jnp.int32)
    w = ii - i0i.astype(jnp.float32)
    i0 = jnp.minimum(jnp.maximum(i0i, 0), GRIDN - 1)
    i1 = jnp.minimum(i0i + 1, GRIDN - 1)
    i1 = jnp.maximum(i1, 0)
    return i0, i1, w


def _t_channel(tvals):
    v = (tvals + jnp.float32(1.0)) * jnp.float32(0.5) * jnp.float32(TSZ - 1)
    f = v.astype(jnp.int32)
    ff = f.astype(jnp.float32)
    d = v - ff
    half = jnp.float32(0.5)
    bump = jnp.where(d > half, 1, jnp.where(d == half, f & 1, 0))
    r = f + bump
    return jnp.minimum(jnp.maximum(r, 0), TSZ - 1)


def _body(vol_h, x_h, y_h, z_h, t_h, consts_h, out_h,
          consts_v, xv, yv, zv, tv, wxv, wyv, wzv,
          i000, i001, i010, i011, i100, i101, i110, i111,
          g000, g001, g010, g011, g100, g101, g110, g111,
          outv, gsem):
    cid = lax.axis_index("c")
    sid = lax.axis_index("s")
    wid = sid * NC + cid

    pltpu.sync_copy(consts_h, consts_v)
    a0x = consts_v[0]
    a0y = consts_v[1]
    a0z = consts_v[2]
    ivx = consts_v[3]
    ivy = consts_v[4]
    ivz = consts_v[5]

    idx_refs = (i000, i001, i010, i011, i100, i101, i110, i111)
    val_refs = (g000, g001, g010, g011, g100, g101, g110, g111)

    def chunk_body(ci, carry):
        base = wid * PPW + ci * C
        pltpu.sync_copy(x_h.at[pl.ds(base, C)], xv)
        pltpu.sync_copy(y_h.at[pl.ds(base, C)], yv)
        pltpu.sync_copy(z_h.at[pl.ds(base, C)], zv)
        pltpu.sync_copy(t_h.at[pl.ds(base, C)], tv)

        def pass1(r, c1):
            for k in range(8):
                off = r * 128 + k * L
                sl = pl.ds(off, L)
                xg = xv[sl]
                yg = yv[sl]
                zg = zv[sl]
                tg = tv[sl]
                ix0, ix1, wx = _axis_index_weight(xg, a0x, ivx)
                iy0, iy1, wy = _axis_index_weight(yg, a0y, ivy)
                iz0, iz1, wz = _axis_index_weight(zg, a0z, ivz)
                tt = _t_channel(tg)
                wxv[sl] = wx
                wyv[sl] = wy
                wzv[sl] = wz
                tc = tt * GRIDN
                sx0, sx1 = ix0, ix1
                sy0, sy1 = _corner_offsets(iy0, iy1, TSZ * GRIDN)
                sz0, sz1 = _corner_offsets(iz0, iz1, GRIDN * TSZ * GRIDN)
                b00 = sz0 + sy0 + tc
                b01 = sz0 + sy1 + tc
                b10 = sz1 + sy0 + tc
                b11 = sz1 + sy1 + tc
                i000[sl] = b00 + sx0
                i001[sl] = b00 + sx1
                i010[sl] = b01 + sx0
                i011[sl] = b01 + sx1
                i100[sl] = b10 + sx0
                i101[sl] = b10 + sx1
                i110[sl] = b11 + sx0
                i111[sl] = b11 + sx1
            return c1

        lax.fori_loop(0, ROWS, pass1, 0)

        handles = []
        for iref, gref in zip(idx_refs, val_refs):
            handles.append(pltpu.async_copy(vol_h.at[iref], gref, gsem))
        for h in handles:
            h.wait()

        def pass2(r, c2):
            for k in range(8):
                off = r * 128 + k * L
                sl = pl.ds(off, L)
                wx = wxv[sl]
                wy = wyv[sl]
                wz = wzv[sl]
                v000 = g000[sl]
                v001 = g001[sl]
                v010 = g010[sl]
                v011 = g011[sl]
                v100 = g100[sl]
                v101 = g101[sl]
                v110 = g110[sl]
                v111 = g111[sl]
                c00 = v000 + wx * (v001 - v000)
                c01 = v010 + wx * (v011 - v010)
                c10 = v100 + wx * (v101 - v100)
                c11 = v110 + wx * (v111 - v110)
                c0 = c00 + wy * (c01 - c00)
                c1 = c10 + wy * (c11 - c10)
                outv[sl] = c0 + wz * (c1 - c0)
            return c2

        lax.fori_loop(0, ROWS, pass2, 0)
        pltpu.sync_copy(outv, out_h.at[pl.ds(base, C)])
        return carry

    lax.fori_loop(0, NCHUNK, chunk_body, 0)


@jax.jit
def _run(vol_flat, x, y, z, t, consts):
    mesh = plsc.VectorSubcoreMesh(
        core_axis_name="c", subcore_axis_name="s", num_cores=NC, num_subcores=NS
    )
    f = pl.kernel(
        _body,
        out_type=jax.ShapeDtypeStruct((NPTS,), jnp.float32),
        mesh=mesh,
        scratch_types=[
            pltpu.VMEM((8, L), jnp.float32),
            pltpu.VMEM((C,), jnp.float32),
            pltpu.VMEM((C,), jnp.float32),
            pltpu.VMEM((C,), jnp.float32),
            pltpu.VMEM((C,), jnp.float32),
            pltpu.VMEM((C,), jnp.float32),
            pltpu.VMEM((C,), jnp.float32),
            pltpu.VMEM((C,), jnp.float32),
        ]
        + [pltpu.VMEM((C,), jnp.int32) for _ in range(8)]
        + [pltpu.VMEM((C,), jnp.float32) for _ in range(8)]
        + [
            pltpu.VMEM((C,), jnp.float32),
            pltpu.SemaphoreType.DMA,
        ],
    )
    return f(vol_flat, x, y, z, t, consts)


def kernel(xyz_sampled, t, aabb, alpha_volume):
    a0 = aabb[0]
    iv = jnp.float32(1.0) / (aabb[1] - aabb[0]) * jnp.float32(2.0)
    consts = jnp.broadcast_to(
        jnp.concatenate([a0, iv, jnp.zeros((2,), jnp.float32)])[:, None], (8, L)
    )
    vol_flat = jnp.transpose(alpha_volume, (0, 1, 3, 2)).reshape(-1)
    x = xyz_sampled[:, 0]
    y = xyz_sampled[:, 1]
    z = xyz_sampled[:, 2]
    return _run(vol_flat, x, y, z, t, consts)


# speedup vs baseline: 14.9693x; 1.2658x over previous
import jax
import jax.numpy as jnp
from jax import lax
from jax.experimental import pallas as pl
from jax.experimental.pallas import tpu as pltpu
from jax.experimental.pallas import tpu_sc as plsc

GRIDN = 128
TSZ = 16
NPTS = 1048576

NC = 2
NS = 16
NW = NC * NS
L = 16

PPW = NPTS // NW
C = 2048
NCHUNK = PPW // C
ROWS = C // 128
NPAIR = NCHUNK // 2 - 1

STRIDE_H = TSZ * GRIDN
STRIDE_D = GRIDN * TSZ * GRIDN


def _axis_index_weight(vals, s, o):
    ii = vals * s + o
    i0i = ii.astype(jnp.int32)
    w = ii - i0i.astype(jnp.float32)
    i0 = jnp.minimum(jnp.maximum(i0i, 0), GRIDN - 1)
    i1 = jnp.maximum(jnp.minimum(i0i + 1, GRIDN - 1), 0)
    return i0, i1, w


def _t_channel(tvals):
    v = (tvals + jnp.float32(1.0)) * jnp.float32(0.5) * jnp.float32(TSZ - 1)
    f = v.astype(jnp.int32)
    d = v - f.astype(jnp.float32)
    half = jnp.float32(0.5)
    bump = jnp.where(d > half, 1, jnp.where(d == half, f & 1, 0))
    r = f + bump
    return jnp.minimum(jnp.maximum(r, 0), TSZ - 1)


def _body(*refs):
    (vol_h, x_h, y_h, z_h, t_h, consts_h, out_h, consts_v) = refs[:8]
    bufA = refs[8:8 + 24]
    bufB = refs[32:32 + 24]
    in_semA, in_semB, gsemA, gsemB = refs[56:60]

    cid = lax.axis_index("c")
    sid = lax.axis_index("s")
    wid = sid * NC + cid
    base_w = wid * PPW

    pltpu.sync_copy(consts_h, consts_v)
    sx = consts_v[0]
    sy = consts_v[1]
    sz = consts_v[2]
    ox = consts_v[3]
    oy = consts_v[4]
    oz = consts_v[5]

    def unpack(buf):
        xv, yv, zv, tv, wxv, wyv, wzv = buf[0:7]
        idx = buf[7:15]
        vals = buf[15:23]
        outv = buf[23]
        return xv, yv, zv, tv, wxv, wyv, wzv, idx, vals, outv

    in_srcs = (x_h, y_h, z_h, t_h)

    def fire_in(c, buf, sem):
        base = base_w + c * C
        for src, dst in zip(in_srcs, buf[0:4]):
            pltpu.async_copy(src.at[pl.ds(base, C)], dst, sem)

    def wait_in(buf, sem):
        for src, dst in zip(in_srcs, buf[0:4]):
            pltpu.make_async_copy(src.at[pl.ds(0, C)], dst, sem).wait()

    def pass1(buf):
        xv, yv, zv, tv, wxv, wyv, wzv, idx, _, _ = unpack(buf)

        def row(r, carry):
            for k in range(8):
                sl = pl.ds(r * 128 + k * L, L)
                ix0, ix1, wx = _axis_index_weight(xv[sl], sx, ox)
                iy0, iy1, wy = _axis_index_weight(yv[sl], sy, oy)
                iz0, iz1, wz = _axis_index_weight(zv[sl], sz, oz)
                tt = _t_channel(tv[sl])
                wxv[sl] = wx
                wyv[sl] = wy
                wzv[sl] = wz
                tc = tt * GRIDN
                b00 = iz0 * STRIDE_D + iy0 * STRIDE_H + tc
                b01 = iz0 * STRIDE_D + iy1 * STRIDE_H + tc
                b10 = iz1 * STRIDE_D + iy0 * STRIDE_H + tc
                b11 = iz1 * STRIDE_D + iy1 * STRIDE_H + tc
                idx[0][sl] = b00 + ix0
                idx[1][sl] = b00 + ix1
                idx[2][sl] = b01 + ix0
                idx[3][sl] = b01 + ix1
                idx[4][sl] = b10 + ix0
                idx[5][sl] = b10 + ix1
                idx[6][sl] = b11 + ix0
                idx[7][sl] = b11 + ix1
            return carry

        lax.fori_loop(0, ROWS, row, 0)

    def fire_g(buf, gsem):
        _, _, _, _, _, _, _, idx, vals, _ = unpack(buf)
        for iref, gref in zip(idx, vals):
            pltpu.async_copy(vol_h.at[iref], gref, gsem)

    def wait_g(buf, gsem):
        _, _, _, _, _, _, _, idx, vals, _ = unpack(buf)
        for iref, gref in zip(idx, vals):
            pltpu.make_async_copy(vol_h.at[iref], gref, gsem).wait()

    def pass2(buf):
        _, _, _, _, wxv, wyv, wzv, _, vals, outv = unpack(buf)

        def row(r, carry):
            for k in range(8):
                sl = pl.ds(r * 128 + k * L, L)
                wx = wxv[sl]
                wy = wyv[sl]
                wz = wzv[sl]
                v000 = vals[0][sl]
                v001 = vals[1][sl]
                v010 = vals[2][sl]
                v011 = vals[3][sl]
                v100 = vals[4][sl]
                v101 = vals[5][sl]
                v110 = vals[6][sl]
                v111 = vals[7][sl]
                c00 = v000 + wx * (v001 - v000)
                c01 = v010 + wx * (v011 - v010)
                c10 = v100 + wx * (v101 - v100)
                c11 = v110 + wx * (v111 - v110)
                c0 = c00 + wy * (c01 - c00)
                c1 = c10 + wy * (c11 - c10)
                outv[sl] = c0 + wz * (c1 - c0)
            return carry

        lax.fori_loop(0, ROWS, row, 0)

    def out_copy(c, buf):
        outv = buf[23]
        pltpu.sync_copy(outv, out_h.at[pl.ds(base_w + c * C, C)])

    last = NCHUNK - 1

    fire_in(0, bufA, in_semA)
    fire_in(1, bufB, in_semB)
    wait_in(bufA, in_semA)
    pass1(bufA)
    fire_g(bufA, gsemA)
    fire_in(2, bufA, in_semA)

    def pair(gi, carry):
        cb = 2 * gi + 1
        wait_in(bufB, in_semB)
        pass1(bufB)
        fire_g(bufB, gsemB)
        fire_in(jnp.minimum(cb + 2, last), bufB, in_semB)
        wait_g(bufA, gsemA)
        pass2(bufA)
        out_copy(2 * gi, bufA)
        ca = 2 * gi + 2
        wait_in(bufA, in_semA)
        pass1(bufA)
        fire_g(bufA, gsemA)
        fire_in(jnp.minimum(ca + 2, last), bufA, in_semA)
        wait_g(bufB, gsemB)
        pass2(bufB)
        out_copy(cb, bufB)
        return carry

    lax.fori_loop(0, NPAIR, pair, 0)

    wait_in(bufB, in_semB)
    pass1(bufB)
    fire_g(bufB, gsemB)
    wait_g(bufA, gsemA)
    pass2(bufA)
    out_copy(NCHUNK - 2, bufA)
    wait_g(bufB, gsemB)
    pass2(bufB)
    out_copy(NCHUNK - 1, bufB)
    wait_in(bufA, in_semA)


def _buf_types():
    return (
        [pltpu.VMEM((C,), jnp.float32) for _ in range(7)]
        + [pltpu.VMEM((C,), jnp.int32) for _ in range(8)]
        + [pltpu.VMEM((C,), jnp.float32) for _ in range(8)]
        + [pltpu.VMEM((C,), jnp.float32)]
    )


@jax.jit
def _run(vol_flat, x, y, z, t, consts):
    mesh = plsc.VectorSubcoreMesh(
        core_axis_name="c", subcore_axis_name="s", num_cores=NC, num_subcores=NS
    )
    f = pl.kernel(
        _body,
        out_type=jax.ShapeDtypeStruct((NPTS,), jnp.float32),
        mesh=mesh,
        scratch_types=[pltpu.VMEM((8, L), jnp.float32)]
        + _buf_types()
        + _buf_types()
        + [pltpu.SemaphoreType.DMA] * 4,
    )
    return f(vol_flat, x, y, z, t, consts)


def kernel(xyz_sampled, t, aabb, alpha_volume):
    a0 = aabb[0]
    iv = jnp.float32(1.0) / (aabb[1] - aabb[0]) * jnp.float32(2.0)
    scale = iv * jnp.float32(0.5 * (GRIDN - 1))
    off = -a0 * scale
    consts = jnp.broadcast_to(
        jnp.concatenate([scale, off, jnp.zeros((2,), jnp.float32)])[:, None], (8, L)
    )
    vol_flat = jnp.transpose(alpha_volume, (0, 1, 3, 2)).reshape(-1)
    x = xyz_sampled[:, 0]
    y = xyz_sampled[:, 1]
    z = xyz_sampled[:, 2]
    return _run(vol_flat, x, y, z, t, consts)


# speedup vs baseline: 15.0728x; 1.0069x over previous
import jax
import jax.numpy as jnp
from jax import lax
from jax.experimental import pallas as pl
from jax.experimental.pallas import tpu as pltpu
from jax.experimental.pallas import tpu_sc as plsc

GRIDN = 128
TSZ = 16
NPTS = 1048576

NC = 2
NS = 16
NW = NC * NS
L = 16

PPW = NPTS // NW
C = 2048
NCHUNK = PPW // C
ROWS = C // 128
NPAIR = NCHUNK // 2 - 1

STRIDE_H = TSZ * GRIDN
STRIDE_D = GRIDN * TSZ * GRIDN


def _axis_index_weight(vals, s, o):
    ii = vals * s + o
    i0i = ii.astype(jnp.int32)
    w = ii - i0i.astype(jnp.float32)
    i0 = jnp.minimum(jnp.maximum(i0i, 0), GRIDN - 1)
    i1 = jnp.maximum(jnp.minimum(i0i + 1, GRIDN - 1), 0)
    return i0, i1, w


def _t_channel(tvals):
    v = (tvals + jnp.float32(1.0)) * jnp.float32(0.5) * jnp.float32(TSZ - 1)
    f = v.astype(jnp.int32)
    d = v - f.astype(jnp.float32)
    half = jnp.float32(0.5)
    bump = jnp.where(d > half, 1, jnp.where(d == half, f & 1, 0))
    r = f + bump
    return jnp.minimum(jnp.maximum(r, 0), TSZ - 1)


def _body(*refs):
    (vol_h, x_h, y_h, z_h, t_h, consts_h, out_h, consts_v) = refs[:8]
    bufA = refs[8:8 + 16]
    bufB = refs[24:24 + 16]
    in_semA, in_semB, gsemA, gsemB = refs[40:44]

    cid = lax.axis_index("c")
    sid = lax.axis_index("s")
    wid = sid * NC + cid
    base_w = wid * PPW

    pltpu.sync_copy(consts_h, consts_v)
    sx = consts_v[0]
    sy = consts_v[1]
    sz = consts_v[2]
    ox = consts_v[3]
    oy = consts_v[4]
    oz = consts_v[5]

    def unpack(buf):
        xv, yv, zv, tv, wxv, wyv, wzv = buf[0:7]
        idx = buf[7:11]
        vals = buf[11:15]
        outv = buf[15]
        return xv, yv, zv, tv, wxv, wyv, wzv, idx, vals, outv

    lanes = lax.broadcasted_iota(jnp.int32, (L,), 0)
    lanes2 = lanes * 2
    lanes2p1 = lanes2 + 1

    in_srcs = (x_h, y_h, z_h, t_h)

    def fire_in(c, buf, sem):
        base = base_w + c * C
        for src, dst in zip(in_srcs, buf[0:4]):
            pltpu.async_copy(src.at[pl.ds(base, C)], dst, sem)

    def wait_in(buf, sem):
        for src, dst in zip(in_srcs, buf[0:4]):
            pltpu.make_async_copy(src.at[pl.ds(0, C)], dst, sem).wait()

    def pass1(buf):
        xv, yv, zv, tv, wxv, wyv, wzv, idx, _, _ = unpack(buf)

        def row(r, carry):
            for k in range(8):
                sl = pl.ds(r * 128 + k * L, L)
                ix0, ix1, wx = _axis_index_weight(xv[sl], sx, ox)
                iy0, iy1, wy = _axis_index_weight(yv[sl], sy, oy)
                iz0, iz1, wz = _axis_index_weight(zv[sl], sz, oz)
                tt = _t_channel(tv[sl])
                wxv[sl] = wx
                wyv[sl] = wy
                wzv[sl] = wz
                tc = tt * GRIDN
                b00 = iz0 * STRIDE_D + iy0 * STRIDE_H + tc
                b01 = iz0 * STRIDE_D + iy1 * STRIDE_H + tc
                b10 = iz1 * STRIDE_D + iy0 * STRIDE_H + tc
                b11 = iz1 * STRIDE_D + iy1 * STRIDE_H + tc
                psl = pl.ds(r * 256 + k * 2 * L, 2 * L)
                plsc.store_scatter(idx[0].at[psl], [lanes2], b00 + ix0)
                plsc.store_scatter(idx[0].at[psl], [lanes2p1], b00 + ix1)
                plsc.store_scatter(idx[1].at[psl], [lanes2], b01 + ix0)
                plsc.store_scatter(idx[1].at[psl], [lanes2p1], b01 + ix1)
                plsc.store_scatter(idx[2].at[psl], [lanes2], b10 + ix0)
                plsc.store_scatter(idx[2].at[psl], [lanes2p1], b10 + ix1)
                plsc.store_scatter(idx[3].at[psl], [lanes2], b11 + ix0)
                plsc.store_scatter(idx[3].at[psl], [lanes2p1], b11 + ix1)
            return carry

        lax.fori_loop(0, ROWS, row, 0)

    def fire_g(buf, gsem):
        _, _, _, _, _, _, _, idx, vals, _ = unpack(buf)
        for iref, gref in zip(idx, vals):
            pltpu.async_copy(vol_h.at[iref], gref, gsem)

    def wait_g(buf, gsem):
        _, _, _, _, _, _, _, idx, vals, _ = unpack(buf)
        for iref, gref in zip(idx, vals):
            pltpu.make_async_copy(vol_h.at[iref], gref, gsem).wait()

    def pass2(buf):
        _, _, _, _, wxv, wyv, wzv, _, vals, outv = unpack(buf)

        def row(r, carry):
            for k in range(8):
                sl = pl.ds(r * 128 + k * L, L)
                wx = wxv[sl]
                wy = wyv[sl]
                wz = wzv[sl]
                psl = pl.ds(r * 256 + k * 2 * L, 2 * L)
                v000 = plsc.load_gather(vals[0].at[psl], [lanes2])
                v001 = plsc.load_gather(vals[0].at[psl], [lanes2p1])
                v010 = plsc.load_gather(vals[1].at[psl], [lanes2])
                v011 = plsc.load_gather(vals[1].at[psl], [lanes2p1])
                v100 = plsc.load_gather(vals[2].at[psl], [lanes2])
                v101 = plsc.load_gather(vals[2].at[psl], [lanes2p1])
                v110 = plsc.load_gather(vals[3].at[psl], [lanes2])
                v111 = plsc.load_gather(vals[3].at[psl], [lanes2p1])
                c00 = v000 + wx * (v001 - v000)
                c01 = v010 + wx * (v011 - v010)
                c10 = v100 + wx * (v101 - v100)
                c11 = v110 + wx * (v111 - v110)
                c0 = c00 + wy * (c01 - c00)
                c1 = c10 + wy * (c11 - c10)
                outv[sl] = c0 + wz * (c1 - c0)
            return carry

        lax.fori_loop(0, ROWS, row, 0)

    def out_copy(c, buf):
        outv = buf[15]
        pltpu.sync_copy(outv, out_h.at[pl.ds(base_w + c * C, C)])

    last = NCHUNK - 1

    fire_in(0, bufA, in_semA)
    fire_in(1, bufB, in_semB)
    wait_in(bufA, in_semA)
    pass1(bufA)
    fire_g(bufA, gsemA)
    fire_in(2, bufA, in_semA)

    def pair(gi, carry):
        cb = 2 * gi + 1
        wait_in(bufB, in_semB)
        pass1(bufB)
        fire_g(bufB, gsemB)
        fire_in(jnp.minimum(cb + 2, last), bufB, in_semB)
        wait_g(bufA, gsemA)
        pass2(bufA)
        out_copy(2 * gi, bufA)
        ca = 2 * gi + 2
        wait_in(bufA, in_semA)
        pass1(bufA)
        fire_g(bufA, gsemA)
        fire_in(jnp.minimum(ca + 2, last), bufA, in_semA)
        wait_g(bufB, gsemB)
        pass2(bufB)
        out_copy(cb, bufB)
        return carry

    lax.fori_loop(0, NPAIR, pair, 0)

    wait_in(bufB, in_semB)
    pass1(bufB)
    fire_g(bufB, gsemB)
    wait_g(bufA, gsemA)
    pass2(bufA)
    out_copy(NCHUNK - 2, bufA)
    wait_g(bufB, gsemB)
    pass2(bufB)
    out_copy(NCHUNK - 1, bufB)
    wait_in(bufA, in_semA)


def _buf_types():
    return (
        [pltpu.VMEM((C,), jnp.float32) for _ in range(7)]
        + [pltpu.VMEM((2 * C,), jnp.int32) for _ in range(4)]
        + [pltpu.VMEM((2 * C,), jnp.float32) for _ in range(4)]
        + [pltpu.VMEM((C,), jnp.float32)]
    )


@jax.jit
def _run(vol_flat, x, y, z, t, consts):
    mesh = plsc.VectorSubcoreMesh(
        core_axis_name="c", subcore_axis_name="s", num_cores=NC, num_subcores=NS
    )
    f = pl.kernel(
        _body,
        out_type=jax.ShapeDtypeStruct((NPTS,), jnp.float32),
        mesh=mesh,
        compiler_params=pltpu.CompilerParams(needs_layout_passes=False),
        scratch_types=[pltpu.VMEM((8, L), jnp.float32)]
        + _buf_types()
        + _buf_types()
        + [pltpu.SemaphoreType.DMA] * 4,
    )
    return f(vol_flat, x, y, z, t, consts)


def kernel(xyz_sampled, t, aabb, alpha_volume):
    a0 = aabb[0]
    iv = jnp.float32(1.0) / (aabb[1] - aabb[0]) * jnp.float32(2.0)
    scale = iv * jnp.float32(0.5 * (GRIDN - 1))
    off = -a0 * scale
    consts = jnp.broadcast_to(
        jnp.concatenate([scale, off, jnp.zeros((2,), jnp.float32)])[:, None], (8, L)
    )
    vol_flat = jnp.transpose(alpha_volume, (0, 1, 3, 2)).reshape(-1)
    x = xyz_sampled[:, 0]
    y = xyz_sampled[:, 1]
    z = xyz_sampled[:, 2]
    return _run(vol_flat, x, y, z, t, consts)
